# Initial kernel scaffold; baseline (speedup 1.0000x reference)
import jax
import jax.numpy as jnp
from jax.experimental import pallas as pl


def kernel(hidden_states, W, b, codevectors):
    raise NotImplementedError("write your pallas kernel here")


# speedup vs baseline: 1.7459x; 1.7459x over previous
import functools

import jax
import jax.numpy as jnp
from jax import lax
from jax.experimental import pallas as pl
from jax.experimental.pallas import tpu as pltpu
from jax.experimental.pallas import tpu_sc as plsc

G = 2
V = 320
DG = 128
H = 512
BT = 8192
TOK_BLK = 1024
N_BLK = BT // TOK_BLK

NC, NS = 2, 16
NW = NC * NS
ROWS = G * BT
ROWS_PER_W = ROWS // NW
CH = 128
NCH = ROWS_PER_W // CH


def _proj_argmax_body(hs_ref, w_ref, b_ref, idx_ref, ppl_ref, counts_scr):
    i = pl.program_id(0)

    @pl.when(i == 0)
    def _init():
        counts_scr[...] = jnp.zeros_like(counts_scr)

    hs = hs_ref[...]
    w = w_ref[...]
    cols = []
    crows = []
    for g in range(G):
        wg = w[g * V:(g + 1) * V, :]
        logits = lax.dot_general(
            hs, wg, (((1,), (1,)), ((), ())),
            preferred_element_type=jnp.float32)
        logits = logits + b_ref[0, g * V:(g + 1) * V][None, :]
        m = jnp.max(logits, axis=1, keepdims=True)
        iota = lax.broadcasted_iota(jnp.int32, logits.shape, 1)
        cand = jnp.where(logits == m, iota, V)
        idx = jnp.min(cand, axis=1)
        onehot = (iota == idx[:, None]).astype(jnp.float32)
        crows.append(jnp.sum(onehot, axis=0, keepdims=True))
        cols.append(idx[:, None] + g * V)
    counts_scr[...] += jnp.concatenate(crows, axis=0)
    idx_ref[...] = jnp.concatenate(cols, axis=1)

    @pl.when(i == N_BLK - 1)
    def _finish():
        p = counts_scr[...] * (1.0 / BT)
        ent = -jnp.sum(p * jnp.log(p + 1e-7), axis=1, keepdims=True)
        ppl_ref[...] = jnp.sum(jnp.exp(ent), axis=0, keepdims=True)


_proj_argmax = pl.pallas_call(
    _proj_argmax_body,
    grid=(N_BLK,),
    in_specs=[
        pl.BlockSpec((TOK_BLK, H), lambda i: (i, 0)),
        pl.BlockSpec((G * V, H), lambda i: (0, 0)),
        pl.BlockSpec((1, G * V), lambda i: (0, 0)),
    ],
    out_specs=[
        pl.BlockSpec((TOK_BLK, G), lambda i: (i, 0)),
        pl.BlockSpec((1, 1), lambda i: (0, 0)),
    ],
    out_shape=[
        jax.ShapeDtypeStruct((BT, G), jnp.int32),
        jax.ShapeDtypeStruct((1, 1), jnp.float32),
    ],
    scratch_shapes=[pltpu.VMEM((G, V), jnp.float32)],
)


@functools.cache
def _make_sc_gather():
    mesh = plsc.VectorSubcoreMesh(core_axis_name="c", subcore_axis_name="s")

    @functools.partial(
        pl.kernel,
        mesh=mesh,
        out_type=jax.ShapeDtypeStruct((ROWS, DG), jnp.float32),
        scratch_types=[
            pltpu.VMEM((NCH, CH), jnp.int32),
            pltpu.VMEM((ROWS_PER_W, DG), jnp.float32),
            pltpu.SemaphoreType.DMA,
        ],
    )
    def _sc_gather(table_hbm, idx_hbm, out_hbm, idx_v, rows_v, sem):
        wid = lax.axis_index("s") * NC + lax.axis_index("c")
        pltpu.sync_copy(idx_hbm.at[pl.ds(wid * NCH, NCH)], idx_v)
        copies = [
            pltpu.async_copy(table_hbm.at[idx_v.at[j]],
                             rows_v.at[pl.ds(j * CH, CH)], sem)
            for j in range(NCH)
        ]
        for c in copies:
            c.wait()
        pltpu.sync_copy(rows_v,
                        out_hbm.at[pl.ds(wid * ROWS_PER_W, ROWS_PER_W)])

    return _sc_gather


def kernel(hidden_states, W, b, codevectors):
    batch, seq, hidden = hidden_states.shape
    hs2 = hidden_states.reshape(batch * seq, hidden)
    idx_pairs, ppl = _proj_argmax(hs2, W, b.reshape(1, G * V))
    idx_flat = idx_pairs.reshape(NW * NCH, CH)
    table = codevectors.reshape(G * V, DG)
    rows = _make_sc_gather()(table, idx_flat)
    codevecs = rows.reshape(batch, seq, G * DG)
    return codevecs, ppl[0, 0]
